# baseline (speedup 1.0000x reference)
import functools

import jax
import jax.numpy as jnp
import numpy as np
from jax import lax
from jax.experimental import pallas as pl
from jax.experimental.pallas import tpu as pltpu
from jax.experimental.pallas import tpu_sc as plsc

N_NODES = 10000
N_EDGES = 160000
N_TRI = 160000
H = 128
NH = 16
DH = 8
NRG = 16
NPAD = 10240
BT = 640
BN = 640
ERW = 32

_f32 = jnp.float32


def _dot(a, b):
    return lax.dot_general(a, b, (((1,), (0,)), ((), ())),
                           preferred_element_type=_f32)


def _ln_relu(z, g, be):
    mu = jnp.mean(z, -1, keepdims=True)
    d = z - mu
    var = jnp.mean(d * d, -1, keepdims=True)
    return jax.nn.relu(d * lax.rsqrt(var + 1e-5) * g + be)


def _np_consts():
    spool = np.zeros((H, NH), np.float32)
    erep = np.zeros((NH, H), np.float32)
    for hh in range(NH):
        spool[hh * DH:(hh + 1) * DH, hh] = 1.0 / np.sqrt(DH)
        erep[hh, hh * DH:(hh + 1) * DH] = 1.0
    rep3 = np.repeat(np.eye(NH, dtype=np.float32), 3, axis=1)
    tile3 = np.zeros((3, NH * 3), np.float32)
    mean3 = np.zeros((NH * 3, 8), np.float32)
    for hh in range(NH):
        for cc in range(3):
            tile3[cc, hh * 3 + cc] = 1.0
            mean3[hh * 3 + cc, cc] = 1.0 / NH
    return (jnp.asarray(spool), jnp.asarray(erep), jnp.asarray(rep3),
            jnp.asarray(tile3), jnp.asarray(mean3))


def _slot_weights(W1):
    D = W1.shape[1]

    def slot(wef, wr):
        z = jnp.zeros((ERW, D), _f32)
        return z.at[0:4].set(wef).at[4:20].set(wr)

    return (slot(W1[0:4], W1[4:20]), slot(W1[20:24], W1[24:40]),
            slot(W1[40:44], W1[44:60]), W1[60:188], W1[188:316], W1[316:444])


def _row(v):
    return v.reshape(1, -1).astype(_f32)


def _full_spec(a):
    nd = a.ndim
    return pl.BlockSpec(a.shape, lambda i, _n=nd: (0,) * _n)


def _sc_gather(table, idx, ch, nk):
    rows, width = idx.shape[0], table.shape[1]
    per = rows // 32
    sup = ch * nk
    assert per % sup == 0
    mesh = plsc.VectorSubcoreMesh(core_axis_name="c", subcore_axis_name="s")

    @functools.partial(
        pl.kernel, mesh=mesh,
        out_type=jax.ShapeDtypeStruct((rows, width), _f32),
        compiler_params=pltpu.CompilerParams(use_tc_tiling_on_sc=False),
        scratch_types=[pltpu.VMEM((sup,), jnp.int32)] +
                      [pltpu.VMEM((ch, width), _f32)] * nk +
                      [pltpu.SemaphoreType.DMA, pltpu.SemaphoreType.DMA])
    def k(tab_hbm, idx_hbm, out_hbm, idx_v, *rest):
        bufs, (gsem, wsem) = rest[:nk], rest[nk:]
        wid = lax.axis_index("s") * 2 + lax.axis_index("c")
        base = wid * per

        @pl.loop(0, per, step=sup)
        def _(off):
            pltpu.sync_copy(idx_hbm.at[pl.ds(base + off, sup)], idx_v)
            gs = [pltpu.async_copy(tab_hbm.at[idx_v.at[pl.ds(b * ch, ch)]],
                                   bufs[b], gsem) for b in range(nk)]
            for g in gs:
                g.wait()
            ws = [pltpu.async_copy(bufs[b],
                                   out_hbm.at[pl.ds(base + off + b * ch, ch)],
                                   wsem) for b in range(nk)]
            for w in ws:
                w.wait()

    return k(table, idx)


def _sc_gather2(taba, idxa, cha, nka, tabb, idxb, chb, nkb):
    rowsa, wa_ = idxa.shape[0], taba.shape[1]
    rowsb, wb_ = idxb.shape[0], tabb.shape[1]
    pera, perb = rowsa // 32, rowsb // 32
    supa, supb = cha * nka, chb * nkb
    assert pera % supa == 0 and perb % supb == 0
    mesh = plsc.VectorSubcoreMesh(core_axis_name="c", subcore_axis_name="s")

    @functools.partial(
        pl.kernel, mesh=mesh,
        out_type=[jax.ShapeDtypeStruct((rowsa, wa_), _f32),
                  jax.ShapeDtypeStruct((rowsb, wb_), _f32)],
        compiler_params=pltpu.CompilerParams(use_tc_tiling_on_sc=False),
        scratch_types=[pltpu.VMEM((supa,), jnp.int32),
                       pltpu.VMEM((supb,), jnp.int32)] +
                      [pltpu.VMEM((cha, wa_), _f32)] * nka +
                      [pltpu.VMEM((chb, wb_), _f32)] * nkb +
                      [pltpu.SemaphoreType.DMA, pltpu.SemaphoreType.DMA])
    def k(ta_hbm, ia_hbm, tb_hbm, ib_hbm, oa_hbm, ob_hbm, ia_v, ib_v, *rest):
        bufsa = rest[:nka]
        bufsb = rest[nka:nka + nkb]
        gsem, wsem = rest[nka + nkb:]
        wid = lax.axis_index("s") * 2 + lax.axis_index("c")

        def run(tab, idx_hbm, out_hbm, idx_v, bufs, per, ch, sup):
            base = wid * per

            @pl.loop(0, per, step=sup)
            def _(off):
                pltpu.sync_copy(idx_hbm.at[pl.ds(base + off, sup)], idx_v)
                gs = [pltpu.async_copy(tab.at[idx_v.at[pl.ds(b * ch, ch)]],
                                       bufs[b], gsem) for b in range(len(bufs))]
                for g in gs:
                    g.wait()
                ws = [pltpu.async_copy(bufs[b],
                                       out_hbm.at[pl.ds(base + off + b * ch, ch)],
                                       wsem) for b in range(len(bufs))]
                for w in ws:
                    w.wait()

        run(ta_hbm, ia_hbm, oa_hbm, ia_v, bufsa, pera, cha, supa)
        run(tb_hbm, ib_hbm, ob_hbm, ib_v, bufsb, perb, chb, supb)

    return k(taba, idxa, tabb, idxb)


def _sc_scatter_add(wa, wb, ids, ch, nk):
    tt, width = wa.shape
    per = tt // 16
    sup = ch * nk
    assert per % sup == 0 and tt % ch == 0
    rows_z = NPAD // 16
    mesh = plsc.VectorSubcoreMesh(core_axis_name="c", subcore_axis_name="s")
    st = jax.ShapeDtypeStruct((NPAD, width), _f32)
    zeros = jnp.zeros((NPAD, width), _f32)
    ids2 = ids.reshape(tt // ch, ch)

    @functools.partial(
        pl.kernel, mesh=mesh, out_type=[st, st],
        compiler_params=pltpu.CompilerParams(use_tc_tiling_on_sc=False),
        scratch_types=[pltpu.VMEM((nk, ch), jnp.int32)] +
                      [pltpu.VMEM((ch, width), _f32)] * nk +
                      [pltpu.VMEM_SHARED((NPAD, width), _f32),
                       pltpu.SemaphoreType.DMA, pltpu.SemaphoreType.DMA])
    def k(wa_hbm, wb_hbm, ids_hbm, z_hbm, o0, o1, idx_v, *rest):
        bufs, (acc, lsem, ssem) = rest[:nk], rest[nk:]
        c = lax.axis_index("c")
        s = lax.axis_index("s")
        r0 = s * rows_z
        pltpu.sync_copy(z_hbm.at[pl.ds(r0, rows_z)], acc.at[pl.ds(r0, rows_z)])
        plsc.subcore_barrier()

        def run(src):
            @pl.loop(0, per, step=sup)
            def _(off):
                row0 = s * per + off
                pltpu.sync_copy(ids_hbm.at[pl.ds(row0 // ch, nk)], idx_v)
                ls = [pltpu.async_copy(src.at[pl.ds(row0 + b * ch, ch)],
                                       bufs[b], lsem) for b in range(nk)]
                for l in ls:
                    l.wait()
                ss = [pltpu.async_copy(bufs[b], acc.at[idx_v.at[b]], ssem,
                                       add=True) for b in range(nk)]
                for sc in ss:
                    sc.wait()

        @pl.when(c == 0)
        def _():
            run(wa_hbm)

        @pl.when(c == 1)
        def _():
            run(wb_hbm)

        plsc.subcore_barrier()

        @pl.when(c == 0)
        def _():
            pltpu.sync_copy(acc.at[pl.ds(r0, rows_z)], o0.at[pl.ds(r0, rows_z)])

        @pl.when(c == 1)
        def _():
            pltpu.sync_copy(acc.at[pl.ds(r0, rows_z)], o1.at[pl.ds(r0, rows_z)])

    return k(wa, wb, ids2, zeros)


def _er_build(edge_feat, xs, xd):
    offs = jnp.linspace(0.0, 10.0, NRG).reshape(1, NRG).astype(_f32)
    coeff = -0.5 / (10.0 / (NRG - 1)) ** 2

    def body(ef_ref, xs_ref, xd_ref, off_ref, out_ref):
        rel = xd_ref[:, :3] - xs_ref[:, :3]
        dist = jnp.sqrt(jnp.sum(rel * rel, -1, keepdims=True) + 1e-12)
        rf = jnp.exp(coeff * (dist - off_ref[...]) ** 2)
        pad = jnp.zeros((rel.shape[0], ERW - 23), _f32)
        out_ref[...] = jnp.concatenate([ef_ref[...], rf, rel, pad], -1)

    grid = N_EDGES // BT
    return pl.pallas_call(
        body, grid=(grid,),
        in_specs=[pl.BlockSpec((BT, 4), lambda i: (i, 0)),
                  pl.BlockSpec((BT, 16), lambda i: (i, 0)),
                  pl.BlockSpec((BT, 16), lambda i: (i, 0)),
                  _full_spec(offs)],
        out_specs=pl.BlockSpec((BT, ERW), lambda i: (i, 0)),
        out_shape=jax.ShapeDtypeStruct((N_EDGES, ERW), _f32),
    )(edge_feat, xs, xd, offs)


def _q_table(hpad, mlp):
    w1, b1, g1, be1 = mlp[0]['W'], _row(mlp[0]['b']), _row(mlp[0]['g']), _row(mlp[0]['be'])
    w2, b2 = mlp[1]['W'], _row(mlp[1]['b'])

    def body(h_ref, w1r, b1r, g1r, be1r, w2r, b2r, out_ref):
        hb = h_ref[...]
        u = _ln_relu(_dot(hb, w1r[...]) + b1r[...], g1r[...], be1r[...])
        q = _dot(u, w2r[...]) + b2r[...]
        out_ref[...] = jnp.concatenate([hb, q], -1)

    grid = NPAD // BN
    args = (hpad, w1, b1, g1, be1, w2, b2)
    return pl.pallas_call(
        body, grid=(grid,),
        in_specs=[pl.BlockSpec((BN, H), lambda i: (i, 0))] +
                 [_full_spec(a) for a in args[1:]],
        out_specs=pl.BlockSpec((BN, 2 * H), lambda i: (i, 0)),
        out_shape=jax.ShapeDtypeStruct((NPAD, 2 * H), _f32),
    )(*args)


def _triplet_pass(ti, hjk, er3, pk, pv, ew, consts, is_x2h):
    spool, erep, rep3, tile3, _ = consts
    W1 = jnp.concatenate([pk[0]['W'], pv[0]['W']], -1)
    b1 = _row(jnp.concatenate([pk[0]['b'], pv[0]['b']], -1))
    g1 = _row(jnp.concatenate([pk[0]['g'], pv[0]['g']], -1))
    be1 = _row(jnp.concatenate([pk[0]['be'], pv[0]['be']], -1))
    we1, we2, we3, whi, whj, whk = _slot_weights(W1)
    w2k, b2k = pk[1]['W'], _row(pk[1]['b'])
    w2v, b2v = pv[1]['W'], _row(pv[1]['b'])
    eww, ewb = _row(ew['W'][:, 0]), _row(ew['b'])
    vw = H if is_x2h else NH * 3
    ow = vw + NH

    def body(ti_ref, hj_ref, hk_ref, eji_ref, eki_ref, ekj_ref,
             we1r, we2r, we3r, whir, whjr, whkr, b1r, g1r, be1r,
             w2kr, b2kr, w2vr, b2vr, ewwr, ewbr, spr, exr, rp3, tl3,
             wa_ref, wb_ref):
        tib = ti_ref[...]
        hi, qi = tib[:, :H], tib[:, H:]
        hj = hj_ref[...]
        hk = hk_ref[...]
        eji = eji_ref[...]
        eki = eki_ref[...]
        ekj = ekj_ref[...]
        shared = _dot(hi, whir[...]) + _dot(ekj, we3r[...]) + b1r[...]
        z_ijk = (shared + _dot(hj, whjr[...]) + _dot(hk, whkr[...])
                 + _dot(eji, we1r[...]) + _dot(eki, we2r[...]))
        z_ikj = (shared + _dot(hk, whjr[...]) + _dot(hj, whkr[...])
                 + _dot(eki, we1r[...]) + _dot(eji, we2r[...]))
        g1b, be1b = g1r[...], be1r[...]

        def head(z):
            uk = _ln_relu(z[:, :H], g1b[:, :H], be1b[:, :H])
            uv = _ln_relu(z[:, H:], g1b[:, H:], be1b[:, H:])
            return (_dot(uk, w2kr[...]) + b2kr[...],
                    _dot(uv, w2vr[...]) + b2vr[...])

        k1, vf1 = head(z_ijk)
        k2, vf2 = head(z_ikj)
        ewwb = ewwr[...]
        ew_ji = jax.nn.sigmoid(
            jnp.sum(eji[:, 4:20] * ewwb, -1, keepdims=True) + ewbr[...])
        ew_ki = jax.nn.sigmoid(
            jnp.sum(eki[:, 4:20] * ewwb, -1, keepdims=True) + ewbr[...])
        if is_x2h:
            v = ((vf1[:, :H] + vf2[:, :H]) * ew_ji
                 + (vf1[:, H:] + vf2[:, H:]) * ew_ki) * 0.5
            exp_mat = exr[...]
        else:
            a = (vf1[:, :NH] + vf2[:, :NH]) * ew_ji * 0.5
            b = (vf1[:, NH:] + vf2[:, NH:]) * ew_ki * 0.5
            rx_ji = eji[:, 20:23]
            rx_ki = eki[:, 20:23]
            rp3b, tl3b = rp3[...], tl3[...]
            v = (_dot(a, rp3b) * _dot(rx_ji, tl3b)
                 + _dot(b, rp3b) * _dot(rx_ki, tl3b))
            exp_mat = rp3b
        spb = spr[...]
        ex1 = jnp.exp(_dot(k1 * qi, spb))
        ex2 = jnp.exp(_dot(k2 * qi, spb))
        wa_ref[...] = jnp.concatenate([_dot(ex1, exp_mat) * v, ex1], -1)
        wb_ref[...] = jnp.concatenate([_dot(ex2, exp_mat) * v, ex2], -1)

    grid = N_TRI // BT
    nb = grid
    weights = (we1, we2, we3, whi, whj, whk, b1, g1, be1,
               w2k, b2k, w2v, b2v, eww, ewb, spool, erep, rep3, tile3)
    in_specs = ([pl.BlockSpec((BT, 2 * H), lambda i: (i, 0)),
                 pl.BlockSpec((BT, H), lambda i: (i, 0)),
                 pl.BlockSpec((BT, H), lambda i, _n=nb: (_n + i, 0)),
                 pl.BlockSpec((BT, ERW), lambda i: (i, 0)),
                 pl.BlockSpec((BT, ERW), lambda i, _n=nb: (_n + i, 0)),
                 pl.BlockSpec((BT, ERW), lambda i, _n=nb: (2 * _n + i, 0))] +
                [_full_spec(a) for a in weights])
    st = jax.ShapeDtypeStruct((N_TRI, ow), _f32)
    return pl.pallas_call(
        body, grid=(grid,),
        in_specs=in_specs,
        out_specs=[pl.BlockSpec((BT, ow), lambda i: (i, 0))] * 2,
        out_shape=[st, st],
    )(ti, hjk, hjk, er3, er3, er3, *weights)


def _combine_x2h(sa, sb, hpad, pno, pq, consts):
    _, erep, _, _, _ = consts
    wn1a, wn1b = pno[0]['W'][:H], pno[0]['W'][H:]
    bn1, gn1, ben1 = _row(pno[0]['b']), _row(pno[0]['g']), _row(pno[0]['be'])
    wn2, bn2 = pno[1]['W'], _row(pno[1]['b'])
    wq1, bq1, gq1, beq1 = pq[0]['W'], _row(pq[0]['b']), _row(pq[0]['g']), _row(pq[0]['be'])
    wq2, bq2 = pq[1]['W'], _row(pq[1]['b'])

    def body(sa_ref, sb_ref, h_ref, exr, wn1ar, wn1br, bn1r, gn1r, ben1r,
             wn2r, bn2r, wq1r, bq1r, gq1r, beq1r, wq2r, bq2r,
             hn_ref, tab_ref):
        sa_b = sa_ref[...]
        sb_b = sb_ref[...]
        exb = exr[...]
        att = (sa_b[:, :H] / (_dot(sa_b[:, H:], exb) + 1e-16)
               + sb_b[:, :H] / (_dot(sb_b[:, H:], exb) + 1e-16)) * 0.5
        hb = h_ref[...]
        z = _dot(att, wn1ar[...]) + _dot(hb, wn1br[...]) + bn1r[...]
        u = _ln_relu(z, gn1r[...], ben1r[...])
        hn = _dot(u, wn2r[...]) + bn2r[...] + hb
        uq = _ln_relu(_dot(hn, wq1r[...]) + bq1r[...], gq1r[...], beq1r[...])
        q2 = _dot(uq, wq2r[...]) + bq2r[...]
        hn_ref[...] = hn
        tab_ref[...] = jnp.concatenate([hn, q2], -1)

    grid = NPAD // BN
    weights = (erep, wn1a, wn1b, bn1, gn1, ben1, wn2, bn2,
               wq1, bq1, gq1, beq1, wq2, bq2)
    return pl.pallas_call(
        body, grid=(grid,),
        in_specs=[pl.BlockSpec((BN, H + NH), lambda i: (i, 0)),
                  pl.BlockSpec((BN, H + NH), lambda i: (i, 0)),
                  pl.BlockSpec((BN, H), lambda i: (i, 0))] +
                 [_full_spec(a) for a in weights],
        out_specs=[pl.BlockSpec((BN, H), lambda i: (i, 0)),
                   pl.BlockSpec((BN, 2 * H), lambda i: (i, 0))],
        out_shape=[jax.ShapeDtypeStruct((NPAD, H), _f32),
                   jax.ShapeDtypeStruct((NPAD, 2 * H), _f32)],
    )(sa, sb, hpad, *weights)


def _combine_h2x(sa, sb, xpad, consts):
    _, _, rep3, _, mean3 = consts
    vw = NH * 3

    def body(sa_ref, sb_ref, x_ref, rp3, m3, out_ref):
        sa_b = sa_ref[...]
        sb_b = sb_ref[...]
        rp3b = rp3[...]
        att = (sa_b[:, :vw] / (_dot(sa_b[:, vw:], rp3b) + 1e-16)
               + sb_b[:, :vw] / (_dot(sb_b[:, vw:], rp3b) + 1e-16)) * 0.5
        out_ref[...] = x_ref[...] + _dot(att, m3[...])

    grid = NPAD // BN
    return pl.pallas_call(
        body, grid=(grid,),
        in_specs=[pl.BlockSpec((BN, vw + NH), lambda i: (i, 0)),
                  pl.BlockSpec((BN, vw + NH), lambda i: (i, 0)),
                  pl.BlockSpec((BN, 8), lambda i: (i, 0)),
                  _full_spec(rep3), _full_spec(mean3)],
        out_specs=pl.BlockSpec((BN, 8), lambda i: (i, 0)),
        out_shape=jax.ShapeDtypeStruct((NPAD, 8), _f32),
    )(sa, sb, xpad, rep3, mean3)


def kernel(h, x, edge_feat, e_w, params, edge_index, id3_i, id3_j, id3_k,
           edgeid_ki, edgeid_ji, edgeid_kj):
    del e_w
    consts = _np_consts()
    hpad = jnp.pad(h, ((0, NPAD - N_NODES), (0, 0)))
    xpad8 = jnp.pad(x, ((0, NPAD - N_NODES), (0, 5)))
    xtab = jnp.pad(x, ((0, 0), (0, 13)))

    srcdst = jnp.concatenate([edge_index[0], edge_index[1]])
    xsd = _sc_gather(xtab, srcdst, 400, 5)
    er = _er_build(edge_feat, xsd[:N_EDGES], xsd[N_EDGES:])
    er3 = _sc_gather(er, jnp.concatenate([edgeid_ji, edgeid_ki, edgeid_kj]),
                     200, 5)

    p1 = params['x2h']
    qtab = _q_table(hpad, p1['hq'])
    ti, hjk = _sc_gather2(qtab, id3_i, 40, 5,
                          hpad, jnp.concatenate([id3_j, id3_k]), 80, 5)
    wa, wb = _triplet_pass(ti, hjk, er3, p1['hk'], p1['hv'], p1['ew'],
                           consts, True)
    sa, sb = _sc_scatter_add(wa, wb, id3_i, 40, 5)
    hnew_pad, tab2 = _combine_x2h(sa, sb, hpad, p1['node_out'], params['h2x']['xq'],
                                  consts)

    p2 = params['h2x']
    ti2, hjk2 = _sc_gather2(tab2, id3_i, 40, 5,
                            hnew_pad, jnp.concatenate([id3_j, id3_k]), 80, 5)
    wa2, wb2 = _triplet_pass(ti2, hjk2, er3, p2['xk'], p2['xv'], p2['ew'],
                             consts, False)
    sa2, sb2 = _sc_scatter_add(wa2, wb2, id3_i, 200, 5)
    xnew = _combine_h2x(sa2, sb2, xpad8, consts)

    return hnew_pad[:N_NODES], xnew[:N_NODES, :3]


# speedup vs baseline: 1.0424x; 1.0424x over previous
import functools

import jax
import jax.numpy as jnp
import numpy as np
from jax import lax
from jax.experimental import pallas as pl
from jax.experimental.pallas import tpu as pltpu
from jax.experimental.pallas import tpu_sc as plsc

N_NODES = 10000
N_EDGES = 160000
N_TRI = 160000
H = 128
NH = 16
DH = 8
NRG = 16
NPAD = 10240
BT = 640
BN = 640
ERW = 32

_f32 = jnp.float32


def _dot(a, b):
    return lax.dot_general(a, b, (((1,), (0,)), ((), ())),
                           preferred_element_type=_f32)


def _ln_relu(z, g, be):
    mu = jnp.mean(z, -1, keepdims=True)
    d = z - mu
    var = jnp.mean(d * d, -1, keepdims=True)
    return jax.nn.relu(d * lax.rsqrt(var + 1e-5) * g + be)


def _cln_relu(z, g, be):
    var = jnp.mean(z * z, -1, keepdims=True)
    return jax.nn.relu(z * lax.rsqrt(var + 1e-5) * g + be)


def _center(w):
    return w - jnp.mean(w, -1, keepdims=True)


def _np_consts():
    spool = np.zeros((H, NH), np.float32)
    erep = np.zeros((NH, H), np.float32)
    for hh in range(NH):
        spool[hh * DH:(hh + 1) * DH, hh] = 1.0 / np.sqrt(DH)
        erep[hh, hh * DH:(hh + 1) * DH] = 1.0
    rep3 = np.repeat(np.eye(NH, dtype=np.float32), 3, axis=1)
    tile3 = np.zeros((3, NH * 3), np.float32)
    mean3 = np.zeros((NH * 3, 8), np.float32)
    for hh in range(NH):
        for cc in range(3):
            tile3[cc, hh * 3 + cc] = 1.0
            mean3[hh * 3 + cc, cc] = 1.0 / NH
    return (jnp.asarray(spool), jnp.asarray(erep), jnp.asarray(rep3),
            jnp.asarray(tile3), jnp.asarray(mean3))


def _slot_weights(W1):
    D = W1.shape[1]

    def slot(wef, wr):
        z = jnp.zeros((ERW, D), _f32)
        return z.at[0:4].set(wef).at[4:20].set(wr)

    return (slot(W1[0:4], W1[4:20]), slot(W1[20:24], W1[24:40]),
            slot(W1[40:44], W1[44:60]), W1[60:188], W1[188:316], W1[316:444])


def _row(v):
    return v.reshape(1, -1).astype(_f32)


def _full_spec(a):
    nd = a.ndim
    return pl.BlockSpec(a.shape, lambda i, _n=nd: (0,) * _n)


def _sc_gather(table, idx, ch, nk):
    rows, width = idx.shape[0], table.shape[1]
    per = rows // 32
    sup = ch * nk
    assert per % sup == 0
    mesh = plsc.VectorSubcoreMesh(core_axis_name="c", subcore_axis_name="s")

    @functools.partial(
        pl.kernel, mesh=mesh,
        out_type=jax.ShapeDtypeStruct((rows, width), _f32),
        compiler_params=pltpu.CompilerParams(use_tc_tiling_on_sc=False),
        scratch_types=[pltpu.VMEM((sup,), jnp.int32)] +
                      [pltpu.VMEM((ch, width), _f32)] * nk +
                      [pltpu.SemaphoreType.DMA, pltpu.SemaphoreType.DMA])
    def k(tab_hbm, idx_hbm, out_hbm, idx_v, *rest):
        bufs, (gsem, wsem) = rest[:nk], rest[nk:]
        wid = lax.axis_index("s") * 2 + lax.axis_index("c")
        base = wid * per

        @pl.loop(0, per, step=sup)
        def _(off):
            pltpu.sync_copy(idx_hbm.at[pl.ds(base + off, sup)], idx_v)
            gs = [pltpu.async_copy(tab_hbm.at[idx_v.at[pl.ds(b * ch, ch)]],
                                   bufs[b], gsem) for b in range(nk)]
            for g in gs:
                g.wait()
            ws = [pltpu.async_copy(bufs[b],
                                   out_hbm.at[pl.ds(base + off + b * ch, ch)],
                                   wsem) for b in range(nk)]
            for w in ws:
                w.wait()

    return k(table, idx)


def _sc_gather2(taba, idxa, cha, nka, tabb, idxb, chb, nkb):
    rowsa, wa_ = idxa.shape[0], taba.shape[1]
    rowsb, wb_ = idxb.shape[0], tabb.shape[1]
    pera, perb = rowsa // 32, rowsb // 32
    supa, supb = cha * nka, chb * nkb
    assert pera % supa == 0 and perb % supb == 0
    mesh = plsc.VectorSubcoreMesh(core_axis_name="c", subcore_axis_name="s")

    @functools.partial(
        pl.kernel, mesh=mesh,
        out_type=[jax.ShapeDtypeStruct((rowsa, wa_), _f32),
                  jax.ShapeDtypeStruct((rowsb, wb_), _f32)],
        compiler_params=pltpu.CompilerParams(use_tc_tiling_on_sc=False),
        scratch_types=[pltpu.VMEM((supa,), jnp.int32),
                       pltpu.VMEM((supb,), jnp.int32)] +
                      [pltpu.VMEM((cha, wa_), _f32)] * nka +
                      [pltpu.VMEM((chb, wb_), _f32)] * nkb +
                      [pltpu.SemaphoreType.DMA, pltpu.SemaphoreType.DMA])
    def k(ta_hbm, ia_hbm, tb_hbm, ib_hbm, oa_hbm, ob_hbm, ia_v, ib_v, *rest):
        bufsa = rest[:nka]
        bufsb = rest[nka:nka + nkb]
        gsem, wsem = rest[nka + nkb:]
        wid = lax.axis_index("s") * 2 + lax.axis_index("c")

        def run(tab, idx_hbm, out_hbm, idx_v, bufs, per, ch, sup):
            base = wid * per

            @pl.loop(0, per, step=sup)
            def _(off):
                pltpu.sync_copy(idx_hbm.at[pl.ds(base + off, sup)], idx_v)
                gs = [pltpu.async_copy(tab.at[idx_v.at[pl.ds(b * ch, ch)]],
                                       bufs[b], gsem) for b in range(len(bufs))]
                for g in gs:
                    g.wait()
                ws = [pltpu.async_copy(bufs[b],
                                       out_hbm.at[pl.ds(base + off + b * ch, ch)],
                                       wsem) for b in range(len(bufs))]
                for w in ws:
                    w.wait()

        run(ta_hbm, ia_hbm, oa_hbm, ia_v, bufsa, pera, cha, supa)
        run(tb_hbm, ib_hbm, ob_hbm, ib_v, bufsb, perb, chb, supb)

    return k(taba, idxa, tabb, idxb)


def _sc_scatter_add(wa, wb, ids, ch, nk):
    tt, width = wa.shape
    per = tt // 16
    sup = ch * nk
    assert per % sup == 0 and tt % ch == 0
    rows_z = NPAD // 16
    mesh = plsc.VectorSubcoreMesh(core_axis_name="c", subcore_axis_name="s")
    st = jax.ShapeDtypeStruct((NPAD, width), _f32)
    zeros = jnp.zeros((NPAD, width), _f32)
    ids2 = ids.reshape(tt // ch, ch)

    @functools.partial(
        pl.kernel, mesh=mesh, out_type=[st, st],
        compiler_params=pltpu.CompilerParams(use_tc_tiling_on_sc=False),
        scratch_types=[pltpu.VMEM((nk, ch), jnp.int32)] +
                      [pltpu.VMEM((ch, width), _f32)] * nk +
                      [pltpu.VMEM_SHARED((NPAD, width), _f32),
                       pltpu.SemaphoreType.DMA, pltpu.SemaphoreType.DMA])
    def k(wa_hbm, wb_hbm, ids_hbm, z_hbm, o0, o1, idx_v, *rest):
        bufs, (acc, lsem, ssem) = rest[:nk], rest[nk:]
        c = lax.axis_index("c")
        s = lax.axis_index("s")
        r0 = s * rows_z
        pltpu.sync_copy(z_hbm.at[pl.ds(r0, rows_z)], acc.at[pl.ds(r0, rows_z)])
        plsc.subcore_barrier()

        def run(src):
            @pl.loop(0, per, step=sup)
            def _(off):
                row0 = s * per + off
                pltpu.sync_copy(ids_hbm.at[pl.ds(row0 // ch, nk)], idx_v)
                ls = [pltpu.async_copy(src.at[pl.ds(row0 + b * ch, ch)],
                                       bufs[b], lsem) for b in range(nk)]
                for l in ls:
                    l.wait()
                ss = [pltpu.async_copy(bufs[b], acc.at[idx_v.at[b]], ssem,
                                       add=True) for b in range(nk)]
                for sc in ss:
                    sc.wait()

        @pl.when(c == 0)
        def _():
            run(wa_hbm)

        @pl.when(c == 1)
        def _():
            run(wb_hbm)

        plsc.subcore_barrier()

        @pl.when(c == 0)
        def _():
            pltpu.sync_copy(acc.at[pl.ds(r0, rows_z)], o0.at[pl.ds(r0, rows_z)])

        @pl.when(c == 1)
        def _():
            pltpu.sync_copy(acc.at[pl.ds(r0, rows_z)], o1.at[pl.ds(r0, rows_z)])

    return k(wa, wb, ids2, zeros)


def _er_build(edge_feat, xs, xd):
    offs = jnp.linspace(0.0, 10.0, NRG).reshape(1, NRG).astype(_f32)
    coeff = -0.5 / (10.0 / (NRG - 1)) ** 2

    def body(ef_ref, xs_ref, xd_ref, off_ref, out_ref):
        rel = xd_ref[:, :3] - xs_ref[:, :3]
        dist = jnp.sqrt(jnp.sum(rel * rel, -1, keepdims=True) + 1e-12)
        rf = jnp.exp(coeff * (dist - off_ref[...]) ** 2)
        pad = jnp.zeros((rel.shape[0], ERW - 23), _f32)
        out_ref[...] = jnp.concatenate([ef_ref[...], rf, rel, pad], -1)

    grid = N_EDGES // BT
    return pl.pallas_call(
        body, grid=(grid,),
        in_specs=[pl.BlockSpec((BT, 4), lambda i: (i, 0)),
                  pl.BlockSpec((BT, 16), lambda i: (i, 0)),
                  pl.BlockSpec((BT, 16), lambda i: (i, 0)),
                  _full_spec(offs)],
        out_specs=pl.BlockSpec((BT, ERW), lambda i: (i, 0)),
        out_shape=jax.ShapeDtypeStruct((N_EDGES, ERW), _f32),
    )(edge_feat, xs, xd, offs)


def _q_table(hpad, mlp):
    w1, b1 = _center(mlp[0]['W']), _center(_row(mlp[0]['b']))
    g1, be1 = _row(mlp[0]['g']), _row(mlp[0]['be'])
    w2, b2 = mlp[1]['W'], _row(mlp[1]['b'])

    def body(h_ref, w1r, b1r, g1r, be1r, w2r, b2r, out_ref):
        hb = h_ref[...]
        u = _cln_relu(_dot(hb, w1r[...]) + b1r[...], g1r[...], be1r[...])
        q = _dot(u, w2r[...]) + b2r[...]
        out_ref[...] = jnp.concatenate([hb, q], -1)

    grid = NPAD // BN
    args = (hpad, w1, b1, g1, be1, w2, b2)
    return pl.pallas_call(
        body, grid=(grid,),
        in_specs=[pl.BlockSpec((BN, H), lambda i: (i, 0))] +
                 [_full_spec(a) for a in args[1:]],
        out_specs=pl.BlockSpec((BN, 2 * H), lambda i: (i, 0)),
        out_shape=jax.ShapeDtypeStruct((NPAD, 2 * H), _f32),
    )(*args)


def _triplet_pass(ti, hjk, er3, pk, pv, ew, consts, is_x2h):
    spool, erep, rep3, tile3, _ = consts
    W1 = jnp.concatenate([_center(pk[0]['W']), _center(pv[0]['W'])], -1)
    b1 = _row(jnp.concatenate([_center(_row(pk[0]['b'])),
                               _center(_row(pv[0]['b']))], -1))
    g1 = _row(jnp.concatenate([pk[0]['g'], pv[0]['g']], -1))
    be1 = _row(jnp.concatenate([pk[0]['be'], pv[0]['be']], -1))
    we1, we2, we3, whi, whj, whk = _slot_weights(W1)
    w2k, b2k = pk[1]['W'], _row(pk[1]['b'])
    w2v, b2v = pv[1]['W'], _row(pv[1]['b'])
    eww, ewb = _row(ew['W'][:, 0]), _row(ew['b'])
    vw = H if is_x2h else NH * 3
    ow = vw + NH

    def body(ti_ref, hj_ref, hk_ref, eji_ref, eki_ref, ekj_ref,
             we1r, we2r, we3r, whir, whjr, whkr, b1r, g1r, be1r,
             w2kr, b2kr, w2vr, b2vr, ewwr, ewbr, spr, exr, rp3, tl3,
             wa_ref, wb_ref):
        tib = ti_ref[...]
        hi, qi = tib[:, :H], tib[:, H:]
        hj = hj_ref[...]
        hk = hk_ref[...]
        eji = eji_ref[...]
        eki = eki_ref[...]
        ekj = ekj_ref[...]
        shared = _dot(hi, whir[...]) + _dot(ekj, we3r[...]) + b1r[...]
        z_ijk = (shared + _dot(hj, whjr[...]) + _dot(hk, whkr[...])
                 + _dot(eji, we1r[...]) + _dot(eki, we2r[...]))
        z_ikj = (shared + _dot(hk, whjr[...]) + _dot(hj, whkr[...])
                 + _dot(eki, we1r[...]) + _dot(eji, we2r[...]))
        g1b, be1b = g1r[...], be1r[...]

        def head(z):
            uk = _cln_relu(z[:, :H], g1b[:, :H], be1b[:, :H])
            uv = _cln_relu(z[:, H:], g1b[:, H:], be1b[:, H:])
            return (_dot(uk, w2kr[...]) + b2kr[...],
                    _dot(uv, w2vr[...]) + b2vr[...])

        k1, vf1 = head(z_ijk)
        k2, vf2 = head(z_ikj)
        ewwb = ewwr[...]
        ew_ji = jax.nn.sigmoid(
            jnp.sum(eji[:, 4:20] * ewwb, -1, keepdims=True) + ewbr[...])
        ew_ki = jax.nn.sigmoid(
            jnp.sum(eki[:, 4:20] * ewwb, -1, keepdims=True) + ewbr[...])
        if is_x2h:
            v = ((vf1[:, :H] + vf2[:, :H]) * ew_ji
                 + (vf1[:, H:] + vf2[:, H:]) * ew_ki) * 0.5
            exp_mat = exr[...]
        else:
            a = (vf1[:, :NH] + vf2[:, :NH]) * ew_ji * 0.5
            b = (vf1[:, NH:] + vf2[:, NH:]) * ew_ki * 0.5
            rx_ji = eji[:, 20:23]
            rx_ki = eki[:, 20:23]
            rp3b, tl3b = rp3[...], tl3[...]
            v = (_dot(a, rp3b) * _dot(rx_ji, tl3b)
                 + _dot(b, rp3b) * _dot(rx_ki, tl3b))
            exp_mat = rp3b
        spb = spr[...]
        ex1 = jnp.exp(_dot(k1 * qi, spb))
        ex2 = jnp.exp(_dot(k2 * qi, spb))
        wa_ref[...] = jnp.concatenate([_dot(ex1, exp_mat) * v, ex1], -1)
        wb_ref[...] = jnp.concatenate([_dot(ex2, exp_mat) * v, ex2], -1)

    grid = N_TRI // BT
    nb = grid
    weights = (we1, we2, we3, whi, whj, whk, b1, g1, be1,
               w2k, b2k, w2v, b2v, eww, ewb, spool, erep, rep3, tile3)
    in_specs = ([pl.BlockSpec((BT, 2 * H), lambda i: (i, 0)),
                 pl.BlockSpec((BT, H), lambda i: (i, 0)),
                 pl.BlockSpec((BT, H), lambda i, _n=nb: (_n + i, 0)),
                 pl.BlockSpec((BT, ERW), lambda i: (i, 0)),
                 pl.BlockSpec((BT, ERW), lambda i, _n=nb: (_n + i, 0)),
                 pl.BlockSpec((BT, ERW), lambda i, _n=nb: (2 * _n + i, 0))] +
                [_full_spec(a) for a in weights])
    st = jax.ShapeDtypeStruct((N_TRI, ow), _f32)
    return pl.pallas_call(
        body, grid=(grid,),
        in_specs=in_specs,
        out_specs=[pl.BlockSpec((BT, ow), lambda i: (i, 0))] * 2,
        out_shape=[st, st],
    )(ti, hjk, hjk, er3, er3, er3, *weights)


def _combine_x2h(sa, sb, hpad, pno, pq, consts):
    _, erep, _, _, _ = consts
    wn1c = _center(pno[0]['W'])
    wn1a, wn1b = wn1c[:H], wn1c[H:]
    bn1, gn1, ben1 = _center(_row(pno[0]['b'])), _row(pno[0]['g']), _row(pno[0]['be'])
    wn2, bn2 = pno[1]['W'], _row(pno[1]['b'])
    wq1, bq1 = _center(pq[0]['W']), _center(_row(pq[0]['b']))
    gq1, beq1 = _row(pq[0]['g']), _row(pq[0]['be'])
    wq2, bq2 = pq[1]['W'], _row(pq[1]['b'])

    def body(sa_ref, sb_ref, h_ref, exr, wn1ar, wn1br, bn1r, gn1r, ben1r,
             wn2r, bn2r, wq1r, bq1r, gq1r, beq1r, wq2r, bq2r,
             hn_ref, tab_ref):
        sa_b = sa_ref[...]
        sb_b = sb_ref[...]
        exb = exr[...]
        att = (sa_b[:, :H] / (_dot(sa_b[:, H:], exb) + 1e-16)
               + sb_b[:, :H] / (_dot(sb_b[:, H:], exb) + 1e-16)) * 0.5
        hb = h_ref[...]
        z = _dot(att, wn1ar[...]) + _dot(hb, wn1br[...]) + bn1r[...]
        u = _cln_relu(z, gn1r[...], ben1r[...])
        hn = _dot(u, wn2r[...]) + bn2r[...] + hb
        uq = _cln_relu(_dot(hn, wq1r[...]) + bq1r[...], gq1r[...], beq1r[...])
        q2 = _dot(uq, wq2r[...]) + bq2r[...]
        hn_ref[...] = hn
        tab_ref[...] = jnp.concatenate([hn, q2], -1)

    grid = NPAD // BN
    weights = (erep, wn1a, wn1b, bn1, gn1, ben1, wn2, bn2,
               wq1, bq1, gq1, beq1, wq2, bq2)
    return pl.pallas_call(
        body, grid=(grid,),
        in_specs=[pl.BlockSpec((BN, H + NH), lambda i: (i, 0)),
                  pl.BlockSpec((BN, H + NH), lambda i: (i, 0)),
                  pl.BlockSpec((BN, H), lambda i: (i, 0))] +
                 [_full_spec(a) for a in weights],
        out_specs=[pl.BlockSpec((BN, H), lambda i: (i, 0)),
                   pl.BlockSpec((BN, 2 * H), lambda i: (i, 0))],
        out_shape=[jax.ShapeDtypeStruct((NPAD, H), _f32),
                   jax.ShapeDtypeStruct((NPAD, 2 * H), _f32)],
    )(sa, sb, hpad, *weights)


def _combine_h2x(sa, sb, xpad, consts):
    _, _, rep3, _, mean3 = consts
    vw = NH * 3

    def body(sa_ref, sb_ref, x_ref, rp3, m3, out_ref):
        sa_b = sa_ref[...]
        sb_b = sb_ref[...]
        rp3b = rp3[...]
        att = (sa_b[:, :vw] / (_dot(sa_b[:, vw:], rp3b) + 1e-16)
               + sb_b[:, :vw] / (_dot(sb_b[:, vw:], rp3b) + 1e-16)) * 0.5
        out_ref[...] = x_ref[...] + _dot(att, m3[...])

    grid = NPAD // BN
    return pl.pallas_call(
        body, grid=(grid,),
        in_specs=[pl.BlockSpec((BN, vw + NH), lambda i: (i, 0)),
                  pl.BlockSpec((BN, vw + NH), lambda i: (i, 0)),
                  pl.BlockSpec((BN, 8), lambda i: (i, 0)),
                  _full_spec(rep3), _full_spec(mean3)],
        out_specs=pl.BlockSpec((BN, 8), lambda i: (i, 0)),
        out_shape=jax.ShapeDtypeStruct((NPAD, 8), _f32),
    )(sa, sb, xpad, rep3, mean3)


def kernel(h, x, edge_feat, e_w, params, edge_index, id3_i, id3_j, id3_k,
           edgeid_ki, edgeid_ji, edgeid_kj):
    del e_w
    consts = _np_consts()
    hpad = jnp.pad(h, ((0, NPAD - N_NODES), (0, 0)))
    xpad8 = jnp.pad(x, ((0, NPAD - N_NODES), (0, 5)))
    xtab = jnp.pad(x, ((0, 0), (0, 13)))

    srcdst = jnp.concatenate([edge_index[0], edge_index[1]])
    xsd = _sc_gather(xtab, srcdst, 400, 5)
    er = _er_build(edge_feat, xsd[:N_EDGES], xsd[N_EDGES:])
    er3 = _sc_gather(er, jnp.concatenate([edgeid_ji, edgeid_ki, edgeid_kj]),
                     200, 5)

    p1 = params['x2h']
    qtab = _q_table(hpad, p1['hq'])
    ti = _sc_gather(qtab, id3_i, 40, 5)
    hjk = _sc_gather(hpad, jnp.concatenate([id3_j, id3_k]), 80, 5)
    wa, wb = _triplet_pass(ti, hjk, er3, p1['hk'], p1['hv'], p1['ew'],
                           consts, True)
    sa, sb = _sc_scatter_add(wa, wb, id3_i, 40, 5)
    hnew_pad, tab2 = _combine_x2h(sa, sb, hpad, p1['node_out'], params['h2x']['xq'],
                                  consts)

    p2 = params['h2x']
    ti2 = _sc_gather(tab2, id3_i, 40, 5)
    hjk2 = _sc_gather(hnew_pad, jnp.concatenate([id3_j, id3_k]), 80, 5)
    wa2, wb2 = _triplet_pass(ti2, hjk2, er3, p2['xk'], p2['xv'], p2['ew'],
                             consts, False)
    sa2, sb2 = _sc_scatter_add(wa2, wb2, id3_i, 200, 5)
    xnew = _combine_h2x(sa2, sb2, xpad8, consts)

    return hnew_pad[:N_NODES], xnew[:N_NODES, :3]
